# baseline (speedup 1.0000x reference)
import functools

import jax
import jax.numpy as jnp
from jax import lax
from jax.experimental import pallas as pl
from jax.experimental.pallas import tpu as pltpu
from jax.experimental.pallas import tpu_sc as plsc

N = 10000
E = 320000
D = 128
ALPHA = 0.2

NC = 2
NS = 16
EP = 327680
EPW = EP // NS
CH = 128
NCH = EPW // CH
IB = 20
NIB = NCH // IB
NP = 10240
RPT = NP // NS


def _sc_segment_sums(ufea, vfea, zrows, *edges):
    mesh = plsc.VectorSubcoreMesh(
        core_axis_name="c", subcore_axis_name="s", num_cores=NC, num_subcores=NS
    )

    @functools.partial(
        pl.kernel,
        out_type=jax.ShapeDtypeStruct((4, NP, D), jnp.float32),
        mesh=mesh,
        scratch_types=[
            pltpu.VMEM_SHARED((NP, D), jnp.float32),
            pltpu.VMEM((IB * CH,), jnp.int32),
            pltpu.VMEM((IB * CH,), jnp.int32),
            pltpu.VMEM((IB * CH,), jnp.int32),
            pltpu.VMEM((IB * CH,), jnp.int32),
            pltpu.VMEM((2, CH, D), jnp.float32),
            pltpu.SemaphoreType.DMA,
            pltpu.SemaphoreType.DMA,
            pltpu.SemaphoreType.DMA,
            pltpu.SemaphoreType.DMA,
            pltpu.SemaphoreType.DMA,
            pltpu.SemaphoreType.DMA,
        ],
    )
    def k(ufea_h, vfea_h, zrows_h,
          uu_d_h, uu_s_h, vv_d_h, vv_s_h, uv_d_h, uv_s_h, vu_d_h, vu_s_h,
          out_h, acc, srcva0, srcva1, dstva0, dstva1, rows,
          gsem0, gsem1, ssem0, ssem1, isem0, isem1):
        srcvas = (srcva0, srcva1)
        dstvas = (dstva0, dstva1)
        core = lax.axis_index("c")
        sub = lax.axis_index("s")
        my_row0 = sub * RPT
        gsems = (gsem0, gsem1)
        ssems = (ssem0, ssem1)

        def zero_my_slice():
            pltpu.sync_copy(zrows_h, acc.at[pl.ds(my_row0, RPT)])

        zero_my_slice()
        plsc.subcore_barrier()

        for kk in range(4):
            fea_h = (ufea_h, vfea_h, vfea_h, ufea_h)[kk]
            dst_h = (uu_d_h, vv_d_h, uv_d_h, vu_d_h)[kk]
            src_h = (uu_s_h, vv_s_h, uv_s_h, vu_s_h)[kk]

            @pl.when(core == kk // 2)
            def _():
                base = sub * EPW
                pltpu.sync_copy(src_h.at[pl.ds(base, IB * CH)], srcva0)
                pltpu.sync_copy(dst_h.at[pl.ds(base, IB * CH)], dstva0)
                pltpu.async_copy(fea_h.at[srcva0.at[pl.ds(0, CH)]],
                                 rows.at[0], gsem0)

                def idx_slice(vas, gb, j):
                    return vas[gb].at[pl.ds(j * CH, CH)]

                def wait_gather(gb, j, b):
                    pltpu.make_async_copy(fea_h.at[idx_slice(srcvas, gb, j)],
                                          rows.at[b], gsems[b]).wait()

                def wait_scatter(gb, j, b):
                    pltpu.make_async_copy(rows.at[b],
                                          acc.at[idx_slice(dstvas, gb, j)],
                                          ssems[b]).wait()

                def process_block(g, gb):
                    @pl.when(g + 1 < NIB)
                    def _():
                        nb = base + (g + 1) * (IB * CH)
                        pltpu.async_copy(src_h.at[pl.ds(nb, IB * CH)],
                                         srcvas[1 - gb], isem0)
                        pltpu.async_copy(dst_h.at[pl.ds(nb, IB * CH)],
                                         dstvas[1 - gb], isem1)

                    def pair(p, _):
                        for b in range(2):
                            j = p * 2 + b
                            if b == 0:
                                @pl.when((j > 0) | (g > 0))
                                def _():
                                    wait_scatter(gb, j, 1)
                            else:
                                wait_scatter(gb, j, 0)

                            @pl.when(j + 1 < IB)
                            def _():
                                pltpu.async_copy(
                                    fea_h.at[idx_slice(srcvas, gb, j + 1)],
                                    rows.at[1 - b], gsems[1 - b])

                            wait_gather(gb, j, b)
                            pltpu.async_copy(rows.at[b],
                                             acc.at[idx_slice(dstvas, gb, j)],
                                             ssems[b], add=True)
                        return 0

                    lax.fori_loop(0, IB // 2, pair, 0)

                    @pl.when(g + 1 < NIB)
                    def _():
                        nb = base + (g + 1) * (IB * CH)
                        pltpu.make_async_copy(src_h.at[pl.ds(nb, IB * CH)],
                                              srcvas[1 - gb], isem0).wait()
                        pltpu.make_async_copy(dst_h.at[pl.ds(nb, IB * CH)],
                                              dstvas[1 - gb], isem1).wait()
                        pltpu.async_copy(fea_h.at[srcvas[1 - gb].at[pl.ds(0, CH)]],
                                         rows.at[0], gsem0)

                def blockpair(q, _):
                    for gb in range(2):
                        process_block(q * 2 + gb, gb)
                    return 0

                lax.fori_loop(0, NIB // 2, blockpair, 0)
                if NIB % 2:
                    process_block(NIB - 1, 0)
                wait_scatter(1, IB - 1, 1)
                plsc.subcore_barrier()
                pltpu.sync_copy(acc.at[pl.ds(my_row0, RPT)],
                                out_h.at[kk, pl.ds(my_row0, RPT)])
                zero_my_slice()
                plsc.subcore_barrier()

    return k(ufea, vfea, zrows, *edges)


BLK = 2000


def _tc_body(agg_ref, ufea_ref, vfea_ref, wg_ref, bg_ref,
             wu1_ref, bu1_ref, wi1_ref, bi1_ref,
             wu_ref, bu_ref, wi_ref, bi_ref, user_ref, item_ref):
    def lrelu(x):
        return jnp.where(x >= 0, x, ALPHA * x)

    def mm(a, b):
        return jnp.dot(a, b, preferred_element_type=jnp.float32)

    h = [lrelu(mm(agg_ref[i], wg_ref[i]) + bg_ref[i]) for i in range(4)]
    uh = mm(h[0], wu1_ref[:D]) + mm(h[2], wu1_ref[D:]) + bu1_ref[...]
    ih = mm(h[1], wi1_ref[:D]) + mm(h[3], wi1_ref[D:]) + bi1_ref[...]
    user_ref[...] = lrelu(mm(uh, wu_ref[:D]) + mm(ufea_ref[...], wu_ref[D:]) + bu_ref[...])
    item_ref[...] = lrelu(mm(ih, wi_ref[:D]) + mm(vfea_ref[...], wi_ref[D:]) + bi_ref[...])


def _tc_epilogue(aggs, ufea, vfea, wg, bg, wu1, bu1, wi1, bi1, wu, bu, wi, bi):
    grid = (N // BLK,)
    full2 = lambda shape: pl.BlockSpec(shape, lambda i: (0, 0))
    full3 = lambda shape: pl.BlockSpec(shape, lambda i: (0, 0, 0))
    return pl.pallas_call(
        _tc_body,
        grid=grid,
        in_specs=[
            pl.BlockSpec((4, BLK, D), lambda i: (0, i, 0)),
            pl.BlockSpec((BLK, D), lambda i: (i, 0)),
            pl.BlockSpec((BLK, D), lambda i: (i, 0)),
            full3((4, D, D)),
            full3((4, 1, D)),
            full2((2 * D, D)),
            full2((1, D)),
            full2((2 * D, D)),
            full2((1, D)),
            full2((2 * D, D)),
            full2((1, D)),
            full2((2 * D, D)),
            full2((1, D)),
        ],
        out_specs=[
            pl.BlockSpec((BLK, D), lambda i: (i, 0)),
            pl.BlockSpec((BLK, D), lambda i: (i, 0)),
        ],
        out_shape=[
            jax.ShapeDtypeStruct((N, D), jnp.float32),
            jax.ShapeDtypeStruct((N, D), jnp.float32),
        ],
    )(aggs, ufea, vfea, wg, bg, wu1, bu1, wi1, bi1, wu, bu, wi, bi)


def kernel(ufea, vfea, UV_adj, VU_adj, uu, vv,
           W_gc1, b_gc1, W_gc2, b_gc2, W_gc3, b_gc3, W_gc4, b_gc4,
           Wu1, bu1, Wi1, bi1, Wu, bu, Wi, bi):
    zrows = jnp.zeros((RPT, D), jnp.float32)
    pidx = jnp.arange(EP - E, dtype=jnp.int32)
    pad_dst = N + pidx % (NP - N)
    pad_src = pidx % N

    def dstrow(e):
        return jnp.concatenate([e[0], pad_dst])

    def srcrow(e):
        return jnp.concatenate([e[1], pad_src])

    aggs = _sc_segment_sums(
        ufea, vfea, zrows,
        dstrow(uu), srcrow(uu), dstrow(vv), srcrow(vv),
        dstrow(UV_adj), srcrow(UV_adj), dstrow(VU_adj), srcrow(VU_adj))
    wg = jnp.stack([W_gc1, W_gc2, W_gc3, W_gc4])
    bg = jnp.stack([b_gc1, b_gc2, b_gc3, b_gc4]).reshape(4, 1, D)
    user, item = _tc_epilogue(
        aggs, ufea, vfea, wg, bg,
        Wu1, bu1.reshape(1, D), Wi1, bi1.reshape(1, D),
        Wu, bu.reshape(1, D), Wi, bi.reshape(1, D),
    )
    return user, item


# speedup vs baseline: 1.2372x; 1.2372x over previous
import functools

import jax
import jax.numpy as jnp
from jax import lax
from jax.experimental import pallas as pl
from jax.experimental.pallas import tpu as pltpu
from jax.experimental.pallas import tpu_sc as plsc

N = 10000
E = 320000
D = 128
ALPHA = 0.2

NC = 2
NS = 16
EPW = E // NS
CH = 128
NCH = 156
TAIL = EPW - NCH * CH
IB = 26
NIB = NCH // IB
NP = 10240
RPT = NP // NS


def _sc_segment_sums(ufea, vfea, zrows, *edges):
    mesh = plsc.VectorSubcoreMesh(
        core_axis_name="c", subcore_axis_name="s", num_cores=NC, num_subcores=NS
    )

    @functools.partial(
        pl.kernel,
        out_type=jax.ShapeDtypeStruct((4, NP, D), jnp.float32),
        mesh=mesh,
        scratch_types=[
            pltpu.VMEM_SHARED((NP, D), jnp.float32),
            pltpu.VMEM((IB * CH,), jnp.int32),
            pltpu.VMEM((IB * CH,), jnp.int32),
            pltpu.VMEM((IB * CH,), jnp.int32),
            pltpu.VMEM((IB * CH,), jnp.int32),
            pltpu.VMEM((2, CH, D), jnp.float32),
            pltpu.SemaphoreType.DMA,
            pltpu.SemaphoreType.DMA,
            pltpu.SemaphoreType.DMA,
            pltpu.SemaphoreType.DMA,
            pltpu.SemaphoreType.DMA,
            pltpu.SemaphoreType.DMA,
        ],
    )
    def k(ufea_h, vfea_h, zrows_h,
          uu_d_h, uu_s_h, vv_d_h, vv_s_h, uv_d_h, uv_s_h, vu_d_h, vu_s_h,
          out_h, acc, srcva0, srcva1, dstva0, dstva1, rows,
          gsem0, gsem1, ssem0, ssem1, isem0, isem1):
        srcvas = (srcva0, srcva1)
        dstvas = (dstva0, dstva1)
        core = lax.axis_index("c")
        sub = lax.axis_index("s")
        my_row0 = sub * RPT
        gsems = (gsem0, gsem1)
        ssems = (ssem0, ssem1)

        def zero_my_slice():
            pltpu.sync_copy(zrows_h, acc.at[pl.ds(my_row0, RPT)])

        zero_my_slice()
        plsc.subcore_barrier()

        for kk in range(4):
            fea_h = (ufea_h, vfea_h, vfea_h, ufea_h)[kk]
            dst_h = (uu_d_h, vv_d_h, uv_d_h, vu_d_h)[kk]
            src_h = (uu_s_h, vv_s_h, uv_s_h, vu_s_h)[kk]

            @pl.when(core == kk // 2)
            def _():
                base = sub * EPW
                pltpu.sync_copy(src_h.at[pl.ds(base, IB * CH)], srcva0)
                pltpu.sync_copy(dst_h.at[pl.ds(base, IB * CH)], dstva0)
                pltpu.async_copy(fea_h.at[srcva0.at[pl.ds(0, CH)]],
                                 rows.at[0], gsem0)

                def idx_slice(vas, gb, j):
                    return vas[gb].at[pl.ds(j * CH, CH)]

                def wait_gather(gb, j, b):
                    pltpu.make_async_copy(fea_h.at[idx_slice(srcvas, gb, j)],
                                          rows.at[b], gsems[b]).wait()

                def wait_scatter(gb, j, b):
                    pltpu.make_async_copy(rows.at[b],
                                          acc.at[idx_slice(dstvas, gb, j)],
                                          ssems[b]).wait()

                def process_block(g, gb):
                    @pl.when(g + 1 < NIB)
                    def _():
                        nb = base + (g + 1) * (IB * CH)
                        pltpu.async_copy(src_h.at[pl.ds(nb, IB * CH)],
                                         srcvas[1 - gb], isem0)
                        pltpu.async_copy(dst_h.at[pl.ds(nb, IB * CH)],
                                         dstvas[1 - gb], isem1)

                    def pair(p, _):
                        for b in range(2):
                            j = p * 2 + b
                            if b == 0:
                                @pl.when((j > 0) | (g > 0))
                                def _():
                                    wait_scatter(gb, j, 1)
                            else:
                                wait_scatter(gb, j, 0)

                            @pl.when(j + 1 < IB)
                            def _():
                                pltpu.async_copy(
                                    fea_h.at[idx_slice(srcvas, gb, j + 1)],
                                    rows.at[1 - b], gsems[1 - b])

                            wait_gather(gb, j, b)
                            pltpu.async_copy(rows.at[b],
                                             acc.at[idx_slice(dstvas, gb, j)],
                                             ssems[b], add=True)
                        return 0

                    lax.fori_loop(0, IB // 2, pair, 0)

                    @pl.when(g + 1 < NIB)
                    def _():
                        nb = base + (g + 1) * (IB * CH)
                        pltpu.make_async_copy(src_h.at[pl.ds(nb, IB * CH)],
                                              srcvas[1 - gb], isem0).wait()
                        pltpu.make_async_copy(dst_h.at[pl.ds(nb, IB * CH)],
                                              dstvas[1 - gb], isem1).wait()
                        pltpu.async_copy(fea_h.at[srcvas[1 - gb].at[pl.ds(0, CH)]],
                                         rows.at[0], gsem0)

                def blockpair(q, _):
                    for gb in range(2):
                        process_block(q * 2 + gb, gb)
                    return 0

                lax.fori_loop(0, NIB // 2, blockpair, 0)
                if NIB % 2:
                    process_block(NIB - 1, 0)
                wait_scatter(1, IB - 1, 1)
                tb = base + NCH * CH
                pltpu.sync_copy(src_h.at[pl.ds(tb, TAIL)],
                                srcva0.at[pl.ds(0, TAIL)])
                pltpu.sync_copy(dst_h.at[pl.ds(tb, TAIL)],
                                dstva0.at[pl.ds(0, TAIL)])
                pltpu.async_copy(fea_h.at[srcva0.at[pl.ds(0, TAIL)]],
                                 rows.at[0, pl.ds(0, TAIL)], gsem0)
                pltpu.make_async_copy(fea_h.at[srcva0.at[pl.ds(0, TAIL)]],
                                      rows.at[0, pl.ds(0, TAIL)], gsem0).wait()
                pltpu.sync_copy(rows.at[0, pl.ds(0, TAIL)],
                                acc.at[dstva0.at[pl.ds(0, TAIL)]], add=True)
                plsc.subcore_barrier()
                pltpu.sync_copy(acc.at[pl.ds(my_row0, RPT)],
                                out_h.at[kk, pl.ds(my_row0, RPT)])
                zero_my_slice()
                plsc.subcore_barrier()

    return k(ufea, vfea, zrows, *edges)


BLK = 2000


def _tc_body(agg_ref, ufea_ref, vfea_ref, wg_ref, bg_ref,
             wu1_ref, bu1_ref, wi1_ref, bi1_ref,
             wu_ref, bu_ref, wi_ref, bi_ref, user_ref, item_ref):
    def lrelu(x):
        return jnp.where(x >= 0, x, ALPHA * x)

    def mm(a, b):
        return jnp.dot(a, b, preferred_element_type=jnp.float32)

    h = [lrelu(mm(agg_ref[i], wg_ref[i]) + bg_ref[i]) for i in range(4)]
    uh = mm(h[0], wu1_ref[:D]) + mm(h[2], wu1_ref[D:]) + bu1_ref[...]
    ih = mm(h[1], wi1_ref[:D]) + mm(h[3], wi1_ref[D:]) + bi1_ref[...]
    user_ref[...] = lrelu(mm(uh, wu_ref[:D]) + mm(ufea_ref[...], wu_ref[D:]) + bu_ref[...])
    item_ref[...] = lrelu(mm(ih, wi_ref[:D]) + mm(vfea_ref[...], wi_ref[D:]) + bi_ref[...])


def _tc_epilogue(aggs, ufea, vfea, wg, bg, wu1, bu1, wi1, bi1, wu, bu, wi, bi):
    grid = (N // BLK,)
    full2 = lambda shape: pl.BlockSpec(shape, lambda i: (0, 0))
    full3 = lambda shape: pl.BlockSpec(shape, lambda i: (0, 0, 0))
    return pl.pallas_call(
        _tc_body,
        grid=grid,
        in_specs=[
            pl.BlockSpec((4, BLK, D), lambda i: (0, i, 0)),
            pl.BlockSpec((BLK, D), lambda i: (i, 0)),
            pl.BlockSpec((BLK, D), lambda i: (i, 0)),
            full3((4, D, D)),
            full3((4, 1, D)),
            full2((2 * D, D)),
            full2((1, D)),
            full2((2 * D, D)),
            full2((1, D)),
            full2((2 * D, D)),
            full2((1, D)),
            full2((2 * D, D)),
            full2((1, D)),
        ],
        out_specs=[
            pl.BlockSpec((BLK, D), lambda i: (i, 0)),
            pl.BlockSpec((BLK, D), lambda i: (i, 0)),
        ],
        out_shape=[
            jax.ShapeDtypeStruct((N, D), jnp.float32),
            jax.ShapeDtypeStruct((N, D), jnp.float32),
        ],
    )(aggs, ufea, vfea, wg, bg, wu1, bu1, wi1, bi1, wu, bu, wi, bi)


def kernel(ufea, vfea, UV_adj, VU_adj, uu, vv,
           W_gc1, b_gc1, W_gc2, b_gc2, W_gc3, b_gc3, W_gc4, b_gc4,
           Wu1, bu1, Wi1, bi1, Wu, bu, Wi, bi):
    zrows = jnp.zeros((RPT, D), jnp.float32)
    aggs = _sc_segment_sums(
        ufea, vfea, zrows,
        uu[0], uu[1], vv[0], vv[1],
        UV_adj[0], UV_adj[1], VU_adj[0], VU_adj[1])
    wg = jnp.stack([W_gc1, W_gc2, W_gc3, W_gc4])
    bg = jnp.stack([b_gc1, b_gc2, b_gc3, b_gc4]).reshape(4, 1, D)
    user, item = _tc_epilogue(
        aggs, ufea, vfea, wg, bg,
        Wu1, bu1.reshape(1, D), Wi1, bi1.reshape(1, D),
        Wu, bu.reshape(1, D), Wi, bi.reshape(1, D),
    )
    return user, item
